# baseline (speedup 1.0000x reference)
import functools

import jax
import jax.numpy as jnp
from jax.experimental import pallas as pl
from jax.experimental.pallas import tpu as pltpu


def _body(x_ref, w_ref, sel_ref, anc_ref, out_ref, *, M: int, A: int):
    scores = jnp.dot(x_ref[...], w_ref[...], preferred_element_type=jnp.float32)

    lane = jax.lax.broadcasted_iota(jnp.int32, scores.shape, 1)
    neg_inf = jnp.float32(-jnp.inf)

    masked = jnp.where(lane < M, scores, neg_inf)
    mmax = jnp.max(masked, axis=1, keepdims=True)
    lse = mmax + jnp.log(jnp.sum(jnp.exp(masked - mmax), axis=1, keepdims=True))

    sel_all = sel_ref[...]
    row_ids = jax.lax.broadcasted_iota(jnp.int32, sel_all.shape, 0)

    acc = jnp.zeros((scores.shape[0], 1), dtype=jnp.float32)
    for a in range(A):
        idx = anc_ref[a]
        sel_row = jnp.sum(jnp.where(row_ids == idx, sel_all, 0.0), axis=0,
                          keepdims=True)
        t = (scores - lse) + jnp.log(sel_row)
        tmax = jnp.max(t, axis=1, keepdims=True)
        acc = acc + tmax + jnp.log(
            jnp.sum(jnp.exp(t - tmax), axis=1, keepdims=True))

    logns0 = jnp.sum(jnp.where(lane == M, scores, 0.0), axis=1, keepdims=True)
    out_ref[...] = -(acc + logns0)


@functools.partial(jax.jit, static_argnames=())
def kernel(x, h0, H, selected_modes, ancestors):
    N, L, Q = x.shape
    M = H.shape[0]
    T = selected_modes.shape[0]
    A = ancestors.shape[0]
    K = L * Q
    LANES = 128

    x2 = x.reshape(N, K)
    w = jnp.concatenate([H.reshape(M, K), h0.reshape(1, K)], axis=0)
    w = jnp.pad(w, ((0, LANES - (M + 1)), (0, 0))).T
    sel = jnp.pad(selected_modes.astype(jnp.float32),
                  ((0, 0), (0, LANES - M)))
    anc = ancestors.astype(jnp.int32)

    BN = 256
    grid = (N // BN,)
    out = pl.pallas_call(
        functools.partial(_body, M=M, A=A),
        grid=grid,
        in_specs=[
            pl.BlockSpec((BN, K), lambda i: (i, 0)),
            pl.BlockSpec((K, LANES), lambda i: (0, 0)),
            pl.BlockSpec((T, LANES), lambda i: (0, 0)),
            pl.BlockSpec(memory_space=pltpu.SMEM),
        ],
        out_specs=pl.BlockSpec((BN, 1), lambda i: (i, 0)),
        out_shape=jax.ShapeDtypeStruct((N, 1), jnp.float32),
        compiler_params=pltpu.CompilerParams(
            dimension_semantics=("parallel",)),
    )(x2, w, sel, anc)
    return out.reshape(N)
